# baseline (speedup 1.0000x reference)
import functools

import jax
import jax.numpy as jnp
from jax import lax
from jax.experimental import pallas as pl
from jax.experimental.pallas import tpu as pltpu, tpu_sc as plsc

N = 10000
D = 128
DH = D // 2
E = 320000
NT = 16
K = 128
CR = E // K
CR_LO = CR // NT
NX = CR - NT * CR_LO
CSZ = CR_LO + 1
NBUF = 4
CT = ((CSZ + NBUF) // NBUF) * NBUF
EPS = 1e-5

NP = 10240
ROWS_PER_TILE = NP // NT
NB_NODE = ROWS_PER_TILE // K


def _stage1_body(x_ref, w_ref, ap_ref, h2_ref, aa_ref):
    h = jnp.dot(x_ref[...], w_ref[...], preferred_element_type=jnp.float32)
    zpad = jnp.zeros((NP - N, DH), jnp.float32)
    h2_ref[0] = jnp.concatenate([h[:, :DH], zpad], axis=0)
    h2_ref[1] = jnp.concatenate([h[:, DH:], zpad], axis=0)
    aap = lax.dot_general(ap_ref[...], h, (((0,), (1,)), ((), ())),
                          preferred_element_type=jnp.float32)
    aa_ref[...] = jnp.concatenate(
        [aap, jnp.zeros((8, NP - N), jnp.float32)], axis=1)


def _stage1(x, W, ap):
    return pl.pallas_call(
        _stage1_body,
        out_shape=[
            jax.ShapeDtypeStruct((2, NP, DH), jnp.float32),
            jax.ShapeDtypeStruct((8, NP), jnp.float32),
        ],
    )(x, W, ap)


def _edge_body(h2_hbm, aa_hbm, e2_hbm, acc_out,
               srcr_v, dstr_v, dst2_v, as_v, ad_v, rows_v, ea_v, idx_v,
               acc_s, den_s, gsem, ssem, isem):
    cid = lax.axis_index("c")
    sid = lax.axis_index("s")

    n_c = jnp.where(sid < NX, CR_LO + 1, CR_LO)
    cb = sid * CR_LO + jnp.minimum(sid, NX)

    pltpu.sync_copy(aa_hbm.at[0], as_v)
    pltpu.sync_copy(aa_hbm.at[1], ad_v)

    def _zero_rows(r, _):
        for q in range(DH // 16):
            rows_v[0, r, pl.ds(16 * q, 16)] = jnp.zeros((16,), jnp.float32)
        return 0
    lax.fori_loop(0, K, _zero_rows, 0)
    for j in range(K // 16):
        ea_v[0, pl.ds(16 * j, 16)] = jnp.zeros((16,), jnp.float32)
    r0 = sid * ROWS_PER_TILE
    for t in range(NB_NODE):
        pltpu.sync_copy(rows_v.at[0], acc_s.at[pl.ds(r0 + K * t, K)])
        pltpu.sync_copy(ea_v.at[0], den_s.at[pl.ds(r0 + K * t, K)])

    plsc.subcore_barrier()

    lane = lax.iota(jnp.int32, 16)
    h_half = h2_hbm.at[cid]

    def _ifetches(c, b):
        eoff = jnp.minimum(cb + jnp.minimum(c, CSZ - 1), CR - 1) * K
        return (pltpu.make_async_copy(e2_hbm.at[0].at[pl.ds(eoff, K)],
                                      srcr_v.at[b], isem.at[b]),
                pltpu.make_async_copy(e2_hbm.at[1].at[pl.ds(eoff, K)],
                                      dstr_v.at[b], isem.at[b]))

    def _gather(c, b):
        return pltpu.make_async_copy(h_half.at[srcr_v.at[b]],
                                     rows_v.at[b], gsem.at[b])

    def _scats(c, b):
        return (pltpu.make_async_copy(rows_v.at[b], acc_s.at[dst2_v.at[b]],
                                      ssem.at[b]),
                pltpu.make_async_copy(ea_v.at[b], den_s.at[dst2_v.at[b]],
                                      ssem.at[b]))

    for d in _ifetches(0, 0):
        d.start()
    for d in _ifetches(1, 1):
        d.start()
    for d in _ifetches(0, 0):
        d.wait()
    _gather(0, 0).start()

    def _chunk(c, _):
        b = lax.rem(c, NBUF)
        bn = lax.rem(c + 1, NBUF)
        b2 = lax.rem(c + 2, NBUF)

        @pl.when(c >= NBUF - 1)
        def _drain():
            for d in _scats(c - (NBUF - 1), bn):
                d.wait()

        @pl.when(c + 1 < CT)
        def _prefetch():
            for d in _ifetches(c + 1, bn):
                d.wait()
            _gather(c + 1, bn).start()

        @pl.when(c + 2 < CT)
        def _ipre():
            for d in _ifetches(c + 2, b2):
                d.start()

        _gather(c, b).wait()

        valid = c < n_c
        for j in range(K // 16):
            s16 = srcr_v[b, pl.ds(16 * j, 16)]
            d16 = dstr_v[b, pl.ds(16 * j, 16)]
            dst2_v[b, pl.ds(16 * j, 16)] = d16
            a = plsc.load_gather(as_v, [s16]) + plsc.load_gather(ad_v, [d16])
            a = jnp.where(a > 0, a, 0.2 * a)
            ea = jnp.where(valid, jnp.exp(a), 0.0)
            ea_v[b, pl.ds(16 * j, 16)] = ea
            for l in range(16):
                s = ea[l]
                r = 16 * j + l
                for q in range(DH // 16):
                    rows_v[b, r, pl.ds(16 * q, 16)] = (
                        rows_v[b, r, pl.ds(16 * q, 16)] * s)

        pltpu.async_copy(rows_v.at[b], acc_s.at[dst2_v.at[b]], ssem.at[b],
                         add=True)
        pltpu.async_copy(ea_v.at[b], den_s.at[dst2_v.at[b]], ssem.at[b],
                         add=True)
        return 0

    lax.fori_loop(0, CT, _chunk, 0)
    for c in range(CT - NBUF + 1, CT):
        for d in _scats(c, c % NBUF):
            d.wait()

    def _selfloop(t, _):
        nb = r0 + K * t
        pltpu.sync_copy(h_half.at[pl.ds(nb, K)], rows_v.at[0])
        for j in range(K // 16):
            node = nb + 16 * j + lane
            a = as_v[pl.ds(nb + 16 * j, 16)] + ad_v[pl.ds(nb + 16 * j, 16)]
            a = jnp.where(a > 0, a, 0.2 * a)
            ea = jnp.where(node < N, jnp.exp(a), 0.0)
            ea_v[0, pl.ds(16 * j, 16)] = ea
            idx_v[pl.ds(16 * j, 16)] = node
            for l in range(16):
                s = ea[l]
                r = 16 * j + l
                for q in range(DH // 16):
                    rows_v[0, r, pl.ds(16 * q, 16)] = (
                        rows_v[0, r, pl.ds(16 * q, 16)] * s)
        pltpu.sync_copy(rows_v.at[0], acc_s.at[idx_v], add=True)
        pltpu.sync_copy(ea_v.at[0], den_s.at[idx_v], add=True)
        return 0

    lax.fori_loop(0, NB_NODE, _selfloop, 0)

    plsc.subcore_barrier()

    out_half = acc_out.at[cid]

    def _normalize(t, _):
        nb = r0 + K * t
        pltpu.sync_copy(acc_s.at[pl.ds(nb, K)], rows_v.at[0])
        pltpu.sync_copy(den_s.at[pl.ds(nb, K)], ea_v.at[0])
        for j in range(K // 16):
            inv = 1.0 / ea_v[0, pl.ds(16 * j, 16)]
            for l in range(16):
                s = inv[l]
                r = 16 * j + l
                for q in range(DH // 16):
                    rows_v[0, r, pl.ds(16 * q, 16)] = (
                        rows_v[0, r, pl.ds(16 * q, 16)] * s)
        pltpu.sync_copy(rows_v.at[0], out_half.at[pl.ds(nb, K)])
        return 0

    lax.fori_loop(0, NB_NODE, _normalize, 0)


@functools.partial(
    pl.kernel,
    out_type=jax.ShapeDtypeStruct((2, NP, DH), jnp.float32),
    mesh=plsc.VectorSubcoreMesh(core_axis_name="c", subcore_axis_name="s"),
    compiler_params=pltpu.CompilerParams(needs_layout_passes=False,
                                         use_tc_tiling_on_sc=False),
    scratch_types=[
        pltpu.VMEM((NBUF, K), jnp.int32),
        pltpu.VMEM((NBUF, K), jnp.int32),
        pltpu.VMEM((NBUF, K), jnp.int32),
        pltpu.VMEM((NP,), jnp.float32),
        pltpu.VMEM((NP,), jnp.float32),
        pltpu.VMEM((NBUF, K, DH), jnp.float32),
        pltpu.VMEM((NBUF, K), jnp.float32),
        pltpu.VMEM((K,), jnp.int32),
        pltpu.VMEM_SHARED((NP, DH), jnp.float32),
        pltpu.VMEM_SHARED((NP,), jnp.float32),
        pltpu.SemaphoreType.DMA((NBUF,)),
        pltpu.SemaphoreType.DMA((NBUF,)),
        pltpu.SemaphoreType.DMA((NBUF,)),
    ],
)
def _edge_kernel(h2_hbm, aa_hbm, e2_hbm, acc_out,
                 srcr_v, dstr_v, dst2_v, as_v, ad_v, rows_v, ea_v, idx_v,
                 acc_s, den_s, gsem, ssem, isem):
    _edge_body(h2_hbm, aa_hbm, e2_hbm, acc_out,
               srcr_v, dstr_v, dst2_v, as_v, ad_v, rows_v, ea_v, idx_v,
               acc_s, den_s, gsem, ssem, isem)


def _stage3_body(acc_ref, bias_ref, bnw_ref, bnb_ref, o_ref):
    val = jnp.concatenate([acc_ref[0, :N], acc_ref[1, :N]], axis=1)
    val = val + bias_ref[...]
    mean = jnp.mean(val, axis=0, keepdims=True)
    ctr = val - mean
    var = jnp.mean(ctr * ctr, axis=0, keepdims=True)
    out = ctr * lax.rsqrt(var + EPS) * bnw_ref[...] + bnb_ref[...]
    o_ref[...] = jnp.maximum(out, 0.0)


def _stage3(acc, bias, bnw, bnb):
    return pl.pallas_call(
        _stage3_body,
        out_shape=jax.ShapeDtypeStruct((N, D), jnp.float32),
    )(acc, bias, bnw, bnb)


def kernel(x, edge_index, edge_attr, W, att_src, att_dst, bias, bn_weight, bn_bias):
    del edge_attr
    ap = jnp.concatenate(
        [att_src[:, None], att_dst[:, None], jnp.zeros((D, 6), jnp.float32)], axis=1)
    h2, aa = _stage1(x, W, ap)
    acc = _edge_kernel(h2, aa, edge_index)
    return _stage3(acc, bias[None, :], bn_weight[None, :], bn_bias[None, :])


# speedup vs baseline: 1.0151x; 1.0151x over previous
import functools

import jax
import jax.numpy as jnp
from jax import lax
from jax.experimental import pallas as pl
from jax.experimental.pallas import tpu as pltpu, tpu_sc as plsc

N = 10000
D = 128
DH = D // 2
E = 320000
NT = 16
K = 128
CR = E // K
CR_LO = CR // NT
NX = CR - NT * CR_LO
CSZ = CR_LO + 1
NBUF = 3
CT = ((CSZ + NBUF) // NBUF) * NBUF
EPS = 1e-5

NP = 10240
ROWS_PER_TILE = NP // NT
NB_NODE = ROWS_PER_TILE // K


def _stage1_body(x_ref, w_ref, ap_ref, h2_ref, aa_ref):
    h = jnp.dot(x_ref[...], w_ref[...], preferred_element_type=jnp.float32)
    zpad = jnp.zeros((NP - N, DH), jnp.float32)
    h2_ref[0] = jnp.concatenate([h[:, :DH], zpad], axis=0)
    h2_ref[1] = jnp.concatenate([h[:, DH:], zpad], axis=0)
    aap = lax.dot_general(ap_ref[...], h, (((0,), (1,)), ((), ())),
                          preferred_element_type=jnp.float32)
    aa_ref[...] = jnp.concatenate(
        [aap, jnp.zeros((8, NP - N), jnp.float32)], axis=1)


def _stage1(x, W, ap):
    return pl.pallas_call(
        _stage1_body,
        out_shape=[
            jax.ShapeDtypeStruct((2, NP, DH), jnp.float32),
            jax.ShapeDtypeStruct((8, NP), jnp.float32),
        ],
    )(x, W, ap)


def _edge_body(h2_hbm, aa_hbm, e2_hbm, pm_hbm, o_hbm,
               srcr_v, dstr_v, dst2_v, as_v, ad_v, rows_v, ea_v,
               pmv, statall_v, stat_v, idx_v, acc_s, den_s, stat_s,
               gsem, ssem, isem):
    cid = lax.axis_index("c")
    sid = lax.axis_index("s")

    n_c = jnp.where(sid < NX, CR_LO + 1, CR_LO)
    cb = sid * CR_LO + jnp.minimum(sid, NX)

    pltpu.sync_copy(aa_hbm.at[0], as_v)
    pltpu.sync_copy(aa_hbm.at[1], ad_v)
    pltpu.sync_copy(pm_hbm.at[cid], pmv)

    def _zero_rows(r, _):
        for q in range(DH // 16):
            rows_v[0, r, pl.ds(16 * q, 16)] = jnp.zeros((16,), jnp.float32)
        return 0
    lax.fori_loop(0, K, _zero_rows, 0)
    for j in range(K // 16):
        ea_v[0, pl.ds(16 * j, 16)] = jnp.zeros((16,), jnp.float32)
    r0 = sid * ROWS_PER_TILE
    for t in range(NB_NODE):
        pltpu.sync_copy(rows_v.at[0], acc_s.at[pl.ds(r0 + K * t, K)])
        pltpu.sync_copy(ea_v.at[0], den_s.at[pl.ds(r0 + K * t, K)])

    plsc.subcore_barrier()

    lane = lax.iota(jnp.int32, 16)
    h_half = h2_hbm.at[cid]

    def _ifetches(c, b):
        eoff = jnp.minimum(cb + jnp.minimum(c, CSZ - 1), CR - 1) * K
        return (pltpu.make_async_copy(e2_hbm.at[0].at[pl.ds(eoff, K)],
                                      srcr_v.at[b], isem.at[b]),
                pltpu.make_async_copy(e2_hbm.at[1].at[pl.ds(eoff, K)],
                                      dstr_v.at[b], isem.at[b]))

    def _gather(c, b):
        return pltpu.make_async_copy(h_half.at[srcr_v.at[b]],
                                     rows_v.at[b], gsem.at[b])

    def _scats(c, b):
        return (pltpu.make_async_copy(rows_v.at[b], acc_s.at[dst2_v.at[b]],
                                      ssem.at[b]),
                pltpu.make_async_copy(ea_v.at[b], den_s.at[dst2_v.at[b]],
                                      ssem.at[b]))

    for d in _ifetches(0, 0):
        d.start()
    for d in _ifetches(1, 1):
        d.start()
    for d in _ifetches(0, 0):
        d.wait()
    _gather(0, 0).start()

    def _chunk(c, _):
        b = lax.rem(c, NBUF)
        bn = lax.rem(c + 1, NBUF)
        b2 = lax.rem(c + 2, NBUF)

        @pl.when(c >= 2)
        def _drain():
            for d in _scats(c - 2, bn):
                d.wait()

        @pl.when(c + 1 < CT)
        def _prefetch():
            for d in _ifetches(c + 1, bn):
                d.wait()
            _gather(c + 1, bn).start()

        @pl.when(c + 2 < CT)
        def _ipre():
            for d in _ifetches(c + 2, b2):
                d.start()

        _gather(c, b).wait()

        valid = c < n_c
        for j in range(K // 16):
            s16 = srcr_v[b, pl.ds(16 * j, 16)]
            d16 = dstr_v[b, pl.ds(16 * j, 16)]
            dst2_v[b, pl.ds(16 * j, 16)] = d16
            a = plsc.load_gather(as_v, [s16]) + plsc.load_gather(ad_v, [d16])
            a = jnp.where(a > 0, a, 0.2 * a)
            ea = jnp.where(valid, jnp.exp(a), 0.0)
            ea_v[b, pl.ds(16 * j, 16)] = ea
            for l in range(16):
                s = ea[l]
                r = 16 * j + l
                for q in range(DH // 16):
                    rows_v[b, r, pl.ds(16 * q, 16)] = (
                        rows_v[b, r, pl.ds(16 * q, 16)] * s)

        pltpu.async_copy(rows_v.at[b], acc_s.at[dst2_v.at[b]], ssem.at[b],
                         add=True)
        pltpu.async_copy(ea_v.at[b], den_s.at[dst2_v.at[b]], ssem.at[b],
                         add=True)
        return 0

    lax.fori_loop(0, CT, _chunk, 0)
    for c in (CT - 2, CT - 1):
        for d in _scats(c, c % NBUF):
            d.wait()

    def _selfloop(t, _):
        nb = r0 + K * t
        pltpu.sync_copy(h_half.at[pl.ds(nb, K)], rows_v.at[0])
        for j in range(K // 16):
            node = nb + 16 * j + lane
            a = as_v[pl.ds(nb + 16 * j, 16)] + ad_v[pl.ds(nb + 16 * j, 16)]
            a = jnp.where(a > 0, a, 0.2 * a)
            ea = jnp.where(node < N, jnp.exp(a), 0.0)
            ea_v[0, pl.ds(16 * j, 16)] = ea
            idx_v[pl.ds(16 * j, 16)] = node
            for l in range(16):
                s = ea[l]
                r = 16 * j + l
                for q in range(DH // 16):
                    rows_v[0, r, pl.ds(16 * q, 16)] = (
                        rows_v[0, r, pl.ds(16 * q, 16)] * s)
        pltpu.sync_copy(rows_v.at[0], acc_s.at[idx_v], add=True)
        pltpu.sync_copy(ea_v.at[0], den_s.at[idx_v], add=True)
        return 0

    lax.fori_loop(0, NB_NODE, _selfloop, 0)

    plsc.subcore_barrier()

    zero16 = jnp.zeros((16,), jnp.float32)

    def _stats(t, carry):
        nb = r0 + K * t
        pltpu.sync_copy(acc_s.at[pl.ds(nb, K)], rows_v.at[0])
        pltpu.sync_copy(den_s.at[pl.ds(nb, K)], ea_v.at[0])
        s1, s2 = carry
        new1, new2 = [], []
        for q in range(DH // 16):
            bq = pmv[0, pl.ds(16 * q, 16)]
            a1, a2 = s1[q], s2[q]
            for j in range(K // 16):
                inv = 1.0 / ea_v[0, pl.ds(16 * j, 16)]
                nvec = nb + 16 * j + lane
                inv = jnp.where(nvec < N, inv, 0.0)
                for l in range(16):
                    s = inv[l]
                    r = 16 * j + l
                    v = rows_v[0, r, pl.ds(16 * q, 16)] * s + bq
                    v = jnp.where(nvec[l] < N, v, 0.0)
                    a1 = a1 + v
                    a2 = a2 + v * v
            new1.append(a1)
            new2.append(a2)
        return tuple(new1), tuple(new2)

    z4 = (zero16,) * (DH // 16)
    s1, s2 = lax.fori_loop(0, NB_NODE, _stats, (z4, z4))
    for q in range(DH // 16):
        stat_v[0, pl.ds(16 * q, 16)] = s1[q]
        stat_v[1, pl.ds(16 * q, 16)] = s2[q]
    pltpu.sync_copy(stat_v, stat_s.at[sid])

    plsc.subcore_barrier()

    pltpu.sync_copy(stat_s, statall_v)
    scales, shifts = [], []
    for q in range(DH // 16):
        t1 = statall_v[0, 0, pl.ds(16 * q, 16)]
        t2 = statall_v[0, 1, pl.ds(16 * q, 16)]
        for u in range(1, NT):
            t1 = t1 + statall_v[u, 0, pl.ds(16 * q, 16)]
            t2 = t2 + statall_v[u, 1, pl.ds(16 * q, 16)]
        mean = t1 * (1.0 / N)
        var = t2 * (1.0 / N) - mean * mean + EPS
        y = plsc.bitcast(jnp.int32(0x5F3759DF) - (plsc.bitcast(var, jnp.int32) >> 1),
                         jnp.float32)
        for _ in range(3):
            y = y * (1.5 - 0.5 * var * y * y)
        sc = pmv[1, pl.ds(16 * q, 16)] * y
        sh = pmv[2, pl.ds(16 * q, 16)] - mean * sc
        cq = pmv[0, pl.ds(16 * q, 16)] * sc + sh
        scales.append(sc)
        shifts.append(cq)

    def _apply(t, _):
        nb = r0 + K * t

        @pl.when(nb < N)
        def _do():
            pltpu.sync_copy(acc_s.at[pl.ds(nb, K)], rows_v.at[0])
            pltpu.sync_copy(den_s.at[pl.ds(nb, K)], ea_v.at[0])
            for j in range(K // 16):
                inv = 1.0 / ea_v[0, pl.ds(16 * j, 16)]
                for l in range(16):
                    s = inv[l]
                    r = 16 * j + l
                    for q in range(DH // 16):
                        v = rows_v[0, r, pl.ds(16 * q, 16)] * (scales[q] * s)
                        v = jnp.maximum(v + shifts[q], 0.0)
                        rows_v[0, r, pl.ds(16 * q, 16)] = v

            @pl.when(nb + K <= N)
            def _full():
                pltpu.sync_copy(rows_v.at[0],
                                o_hbm.at[pl.ds(nb, K), pl.ds(cid * DH, DH)])

            @pl.when(nb + K > N)
            def _part():
                pltpu.sync_copy(rows_v.at[0].at[pl.ds(0, N % K)],
                                o_hbm.at[pl.ds(nb, N % K), pl.ds(cid * DH, DH)])
        return 0

    lax.fori_loop(0, NB_NODE, _apply, 0)


@functools.partial(
    pl.kernel,
    out_type=jax.ShapeDtypeStruct((N, D), jnp.float32),
    mesh=plsc.VectorSubcoreMesh(core_axis_name="c", subcore_axis_name="s"),
    compiler_params=pltpu.CompilerParams(needs_layout_passes=False,
                                         use_tc_tiling_on_sc=False),
    scratch_types=[
        pltpu.VMEM((NBUF, K), jnp.int32),
        pltpu.VMEM((NBUF, K), jnp.int32),
        pltpu.VMEM((NBUF, K), jnp.int32),
        pltpu.VMEM((NP,), jnp.float32),
        pltpu.VMEM((NP,), jnp.float32),
        pltpu.VMEM((NBUF, K, DH), jnp.float32),
        pltpu.VMEM((NBUF, K), jnp.float32),
        pltpu.VMEM((8, DH), jnp.float32),
        pltpu.VMEM((NT, 2, DH), jnp.float32),
        pltpu.VMEM((2, DH), jnp.float32),
        pltpu.VMEM((K,), jnp.int32),
        pltpu.VMEM_SHARED((NP, DH), jnp.float32),
        pltpu.VMEM_SHARED((NP,), jnp.float32),
        pltpu.VMEM_SHARED((NT, 2, DH), jnp.float32),
        pltpu.SemaphoreType.DMA((NBUF,)),
        pltpu.SemaphoreType.DMA((NBUF,)),
        pltpu.SemaphoreType.DMA((NBUF,)),
    ],
)
def _edge_kernel(h2_hbm, aa_hbm, e2_hbm, pm_hbm, o_hbm,
                 srcr_v, dstr_v, dst2_v, as_v, ad_v, rows_v, ea_v,
                 pmv, statall_v, stat_v, idx_v, acc_s, den_s, stat_s,
                 gsem, ssem, isem):
    _edge_body(h2_hbm, aa_hbm, e2_hbm, pm_hbm, o_hbm,
               srcr_v, dstr_v, dst2_v, as_v, ad_v, rows_v, ea_v,
               pmv, statall_v, stat_v, idx_v, acc_s, den_s, stat_s,
               gsem, ssem, isem)


def kernel(x, edge_index, edge_attr, W, att_src, att_dst, bias, bn_weight, bn_bias):
    del edge_attr
    ap = jnp.concatenate(
        [att_src[:, None], att_dst[:, None], jnp.zeros((D, 6), jnp.float32)], axis=1)
    h2, aa = _stage1(x, W, ap)
    pm = jnp.concatenate([
        jnp.stack([bias.reshape(2, DH), bn_weight.reshape(2, DH),
                   bn_bias.reshape(2, DH)], axis=1),
        jnp.zeros((2, 5, DH), jnp.float32)], axis=1)
    return _edge_kernel(h2, aa, edge_index, pm)
